# baseline (speedup 1.0000x reference)
import functools
from math import sqrt

import jax
import jax.numpy as jnp
from jax.experimental import pallas as pl
from jax.experimental.pallas import tpu as pltpu


def _fused_body(values_ref, alpha_ref, gain_ref, gamma_ref, u_ref, v_ref,
                lnw_ref, lnb_ref, out_ref, score_ref, *, k, f, d):
    scale = 1.0 / sqrt(f)
    v2 = values_ref[...]
    nb = v2.shape[0]

    @pl.when(pl.program_id(1) == 0)
    def _():
        sq = v2 * v2
        for hh in range(2):
            energy = jnp.mean(sq[:, :, hh * d:(hh + 1) * d], axis=-1)
            rms = jnp.maximum(
                jnp.sqrt(jnp.mean(energy, axis=-1, keepdims=True)), 1e-6)
            gain = jnp.log1p(jnp.exp(gain_ref[hh]))
            sc = (energy / rms) * gain
            mu = jnp.mean(sc, axis=-1, keepdims=True)
            var = jnp.mean((sc - mu) ** 2, axis=-1, keepdims=True)
            score_ref[hh] = ((sc - mu) / jnp.sqrt(var + 1e-5) * lnw_ref[0]
                             + lnb_ref[0])

    rows = []
    for hh in range(2):
        score = score_ref[hh]
        bil = jnp.dot(u_ref[hh], v_ref[hh],
                      preferred_element_type=jnp.float32)
        g = gamma_ref[hh]
        rows += [bil + g + score[b][None, :] for b in range(nb)]
    rows.append(alpha_ref[0] * scale)
    rows.append(alpha_ref[1] * scale)
    x = jnp.concatenate(rows, axis=0)

    bits = jax.lax.bitcast_convert_type(x, jnp.int32)
    sign = jax.lax.shift_right_arithmetic(bits, 31)
    kb = jnp.bfloat16(k)
    one_b = jnp.bfloat16(1)
    zero_b = jnp.bfloat16(0)
    hi16 = (jax.lax.shift_right_arithmetic(bits, 16)
            ^ (sign & jnp.int32(0x7FFF))).astype(jnp.int16)

    def count_hi(cand):
        return jnp.sum(jnp.where(hi16 >= cand, one_b, zero_b), axis=-1,
                       keepdims=True, dtype=jnp.bfloat16)

    cnt = count_hi(jnp.int16(0))
    prefix_hi = jnp.where(cnt >= kb, jnp.int16(0), jnp.int16(-32768))
    for j in range(14, -1, -1):
        cand = prefix_hi | jnp.int16(1 << j)
        prefix_hi = jnp.where(count_hi(cand) >= kb, cand, prefix_hi)
    ulo = (bits.astype(jnp.int16) ^ sign.astype(jnp.int16)
           ^ jnp.int16(-32768))
    eq = hi16 == prefix_hi
    cnt_above = jnp.sum(jnp.where(hi16 > prefix_hi, one_b, zero_b),
                        axis=-1, keepdims=True, dtype=jnp.bfloat16)
    cnt = cnt_above + jnp.sum(
        jnp.where(eq & (ulo >= jnp.int16(0)), one_b, zero_b), axis=-1,
        keepdims=True, dtype=jnp.bfloat16)
    prefix_lo = jnp.where(cnt >= kb, jnp.int16(0), jnp.int16(-32768))
    for j in range(14, 7, -1):
        cand = prefix_lo | jnp.int16(1 << j)
        cnt = cnt_above + jnp.sum(
            jnp.where(eq & (ulo >= cand), one_b, zero_b), axis=-1,
            keepdims=True, dtype=jnp.bfloat16)
        prefix_lo = jnp.where(cnt >= kb, cand, prefix_lo)
    keep = (hi16 > prefix_hi) | (eq & (ulo >= prefix_lo))

    m = jnp.max(x, axis=-1, keepdims=True)
    p = jnp.where(keep, jnp.exp(x - m), 0.0)
    z = jnp.sum(p, axis=-1, keepdims=True)
    a = p / z

    ts = a.shape[0] // (2 * nb + 2)
    outs = []
    for b in range(nb):
        cols = []
        for hh in range(2):
            attn = (a[(nb * hh + b) * ts:(nb * hh + b + 1) * ts]
                    + a[(2 * nb + hh) * ts:(2 * nb + hh + 1) * ts])
            cols.append(jnp.dot(attn, v2[b, :, hh * d:(hh + 1) * d],
                                preferred_element_type=jnp.float32))
        outs.append(jnp.concatenate(cols, axis=-1))
    out_ref[...] = jnp.stack(outs, axis=0)


def kernel(values, alpha, temp, gamma_hs, U, V, ln_w, ln_b):
    B, F, H, D = values.shape
    _, S, _ = alpha.shape
    R = U.shape[-1]
    TS = 256
    k = max(1, int(0.1 * F))

    vflat = values.reshape(B, F, H * D)
    temp_b = jnp.broadcast_to(temp[:, None], (H, 1, F))

    out = pl.pallas_call(
        functools.partial(_fused_body, k=k, f=F, d=D),
        grid=(H // 2, S // TS),
        in_specs=[
            pl.BlockSpec((B, F, 2 * D), lambda h, s: (0, 0, h)),
            pl.BlockSpec((2, TS, F), lambda h, s: (h, s, 0)),
            pl.BlockSpec((2, 1, F), lambda h, s: (h, 0, 0)),
            pl.BlockSpec((2, TS, 1), lambda h, s: (h, s, 0)),
            pl.BlockSpec((2, TS, R), lambda h, s: (h, s, 0)),
            pl.BlockSpec((2, R, F), lambda h, s: (h, 0, 0)),
            pl.BlockSpec((1, 1, F), lambda h, s: (0, 0, 0)),
            pl.BlockSpec((1, 1, F), lambda h, s: (0, 0, 0)),
        ],
        out_specs=pl.BlockSpec((B, TS, 2 * D), lambda h, s: (0, s, h)),
        out_shape=jax.ShapeDtypeStruct((B, S, H * D), jnp.float32),
        scratch_shapes=[pltpu.VMEM((2, B, F), jnp.float32)],
    )(vflat, alpha, temp_b, gamma_hs, U, V,
      ln_w.reshape(1, 1, F), ln_b.reshape(1, 1, F))
    return out.reshape(B, S, H, D)
